# baseline (speedup 1.0000x reference)
import functools

import jax
import jax.numpy as jnp
from jax import lax
from jax.experimental import pallas as pl
from jax.experimental.pallas import tpu as pltpu
from jax.experimental.pallas import tpu_sc as plsc


def _tc_reduce_body(x_ref, attr_ref, a_ref, *, num_patches, patch, m_blk, mm):
    i = pl.program_id(0)
    for j in range(m_blk):
        b = (i * m_blk + j) // mm
        s = jnp.sign(x_ref[b])
        v = jnp.maximum(attr_ref[j] * s[None, :], 0.0)
        psum = v.reshape(num_patches, patch, v.shape[-1]).sum(axis=(1, 2))
        a_ref[j, 0] = psum


def _sc_reduce_body(x_hbm, attr_hbm, out_hbm, xv, sgn, buf0, buf1, outv,
                    sem0, sem1, *, bm_base, mm, num_patches, patch, ppw,
                    rows, n, nb):
    nc = 2
    wid = lax.axis_index("s") * nc + lax.axis_index("c")
    p0 = wid * ppw

    pltpu.sync_copy(x_hbm, xv)
    for bi in range(nb):
        def sgn_body(c, carry):
            sgn[bi, pl.ds(c * 16, 16)] = jnp.sign(xv[bi, pl.ds(c * 16, 16)])
            return carry
        lax.fori_loop(0, n // 16, sgn_body, jnp.float32(0.0))

    halves = patch // rows
    bufs = (buf0, buf1)
    sems = (sem0, sem1)
    total = ppw * halves

    def chunk_src(i):
        pf = p0 + (i // halves)
        bm = bm_base + pf // num_patches
        f0 = (pf % num_patches) * patch + (i % halves) * rows
        return attr_hbm.at[bm, pl.ds(f0, rows)]

    copies = [None, None]
    copies[0] = pltpu.async_copy(chunk_src(0), bufs[0], sems[0])
    accs = (jnp.zeros((16,), jnp.float32),) * 4
    for i in range(total):
        if i + 1 < total:
            copies[(i + 1) % 2] = pltpu.async_copy(
                chunk_src(i + 1), bufs[(i + 1) % 2], sems[(i + 1) % 2])
        copies[i % 2].wait()
        buf = bufs[i % 2]
        pf = p0 + (i // halves)
        b = (bm_base + pf // num_patches) // mm

        def cbody(c, accs):
            a0, a1, a2, a3 = accs
            sg = sgn[b, pl.ds(c * 16, 16)]
            for r in range(rows):
                v = jnp.maximum(buf[r, pl.ds(c * 16, 16)] * sg, 0.0)
                if r % 4 == 0:
                    a0 = a0 + v
                elif r % 4 == 1:
                    a1 = a1 + v
                elif r % 4 == 2:
                    a2 = a2 + v
                else:
                    a3 = a3 + v
            return (a0, a1, a2, a3)
        accs = lax.fori_loop(0, n // 16, cbody, accs)
        if i % halves == halves - 1:
            p_local = i // halves
            outv[pl.ds(p_local * 16, 16)] = (accs[0] + accs[1]) + (accs[2] + accs[3])
            accs = (jnp.zeros((16,), jnp.float32),) * 4

    pltpu.sync_copy(outv, out_hbm.at[pl.ds(wid * ppw * 16, ppw * 16)])


def _tail_body(a_tc_ref, a_sc_ref, xr_ref, wt_ref, biasr_ref, out_ref, *,
               num_patches, patch):
    P = num_patches
    a_sc = jnp.sum(a_sc_ref[:], axis=-1)
    a_full = jnp.concatenate([a_tc_ref[:], a_sc], axis=0)
    T = xr_ref[:] * wt_ref[:]
    N = T.shape[-1]
    n_iota = jax.lax.broadcasted_iota(jnp.int32, (N, P), 0)
    p_iota = jax.lax.broadcasted_iota(jnp.int32, (N, P), 1)
    ind = ((n_iota // patch) == p_iota).astype(jnp.float32)
    pd = jnp.dot(T, ind, preferred_element_type=jnp.float32)

    a2 = a_full
    ap = a2[:, :, None]
    aq = a2[:, None, :]
    qi = jax.lax.broadcasted_iota(jnp.int32, (a2.shape[0], P, P), 2)
    pi = jax.lax.broadcasted_iota(jnp.int32, (a2.shape[0], P, P), 1)
    beats = (aq > ap) | ((aq == ap) & (qi < pi))
    rank = jnp.sum(beats.astype(jnp.float32), axis=-1)
    wgt = jnp.float32(P) - rank

    S = jnp.sum(wgt * pd, axis=-1, keepdims=True)
    biasr = biasr_ref[:]
    inf0 = jnp.sum(pd, axis=-1, keepdims=True) + biasr
    dx = jnp.float32(1.0 / (P + 2))
    out_ref[:] = dx * (0.5 * (1.0 + biasr / inf0)
                       + (jnp.float32(P) * inf0 - S) / inf0)


def kernel(x, attr, mask, W, bias):
    B, M, F, N = attr.shape
    patch = int(F * 0.0625)
    P = F // patch
    BM = B * M

    K_SC = 2
    BM_TC = BM - K_SC
    NW = 32
    PPW = K_SC * P // NW
    ROWS = 32

    attr3 = attr.reshape(BM, F, N)

    M_BLK = 2
    a_tc = pl.pallas_call(
        functools.partial(_tc_reduce_body, num_patches=P, patch=patch,
                          m_blk=M_BLK, mm=M),
        grid=(BM_TC // M_BLK,),
        in_specs=[
            pl.BlockSpec((B, N), lambda i: (0, 0)),
            pl.BlockSpec((M_BLK, F, N), lambda i: (i, 0, 0)),
        ],
        out_specs=pl.BlockSpec((M_BLK, 1, P), lambda i: (i, 0, 0)),
        out_shape=jax.ShapeDtypeStruct((BM_TC, 1, P), jnp.float32),
    )(x, attr3)

    mesh = plsc.VectorSubcoreMesh(core_axis_name="c", subcore_axis_name="s")
    sc_out = pl.kernel(
        functools.partial(_sc_reduce_body, bm_base=BM_TC, mm=M, num_patches=P,
                          patch=patch, ppw=PPW, rows=ROWS, n=N, nb=B),
        out_type=jax.ShapeDtypeStruct((K_SC * P * 16,), jnp.float32),
        mesh=mesh,
        scratch_types=[
            pltpu.VMEM((B, N), jnp.float32),
            pltpu.VMEM((B, N), jnp.float32),
            pltpu.VMEM((ROWS, N), jnp.float32),
            pltpu.VMEM((ROWS, N), jnp.float32),
            pltpu.VMEM((PPW * 16,), jnp.float32),
            pltpu.SemaphoreType.DMA,
            pltpu.SemaphoreType.DMA,
        ],
    )(x, attr3)

    a_sc3 = sc_out.reshape(K_SC, P, 16)

    xr = jnp.repeat(x, M, axis=0)
    wt = jnp.tile(W.T, (B, 1))
    biasr = jnp.tile(bias, B).reshape(BM, 1)

    out_flat = pl.pallas_call(
        functools.partial(_tail_body, num_patches=P, patch=patch),
        out_shape=jax.ShapeDtypeStruct((BM, 1), jnp.float32),
    )(a_tc.reshape(BM_TC, P), a_sc3, xr, wt, biasr)
    return out_flat.reshape(B, M)


# speedup vs baseline: 1.1776x; 1.1776x over previous
import functools

import jax
import jax.numpy as jnp
from jax import lax
from jax.experimental import pallas as pl
from jax.experimental.pallas import tpu as pltpu
from jax.experimental.pallas import tpu_sc as plsc


def _tc_reduce_body(x_ref, attr0_ref, attr1_ref, a0_ref, a1_ref, *,
                    num_patches, patch, m_blk, mm, half):
    i = pl.program_id(0)
    for k, (attr_ref, a_ref) in enumerate(((attr0_ref, a0_ref),
                                           (attr1_ref, a1_ref))):
        for j in range(m_blk):
            b = (k * half + i * m_blk + j) // mm
            s = jnp.sign(x_ref[b])
            v = jnp.maximum(attr_ref[j] * s[None, :], 0.0)
            psum = v.reshape(num_patches, patch, v.shape[-1]).sum(axis=(1, 2))
            a_ref[j, 0] = psum


def _sc_reduce_body(x_hbm, attr_hbm, out_hbm, xv, sgn, buf0, buf1, outv,
                    sem0, sem1, *, bm_base, mm, num_patches, patch, ppw,
                    rows, n, nb):
    nc = 2
    wid = lax.axis_index("s") * nc + lax.axis_index("c")
    p0 = wid * ppw

    pltpu.sync_copy(x_hbm, xv)
    for bi in range(nb):
        def sgn_body(c, carry):
            sgn[bi, pl.ds(c * 16, 16)] = jnp.sign(xv[bi, pl.ds(c * 16, 16)])
            return carry
        lax.fori_loop(0, n // 16, sgn_body, jnp.float32(0.0))

    halves = patch // rows
    bufs = (buf0, buf1)
    sems = (sem0, sem1)
    total = ppw * halves

    def chunk_src(i):
        pf = p0 + (i // halves)
        bm = bm_base + pf // num_patches
        f0 = (pf % num_patches) * patch + (i % halves) * rows
        return attr_hbm.at[bm, pl.ds(f0, rows)]

    copies = [None, None]
    copies[0] = pltpu.async_copy(chunk_src(0), bufs[0], sems[0])
    accs = (jnp.zeros((16,), jnp.float32),) * 4
    for i in range(total):
        if i + 1 < total:
            copies[(i + 1) % 2] = pltpu.async_copy(
                chunk_src(i + 1), bufs[(i + 1) % 2], sems[(i + 1) % 2])
        copies[i % 2].wait()
        buf = bufs[i % 2]
        pf = p0 + (i // halves)
        b = (bm_base + pf // num_patches) // mm

        def cbody(c, accs):
            a0, a1, a2, a3 = accs
            sg = sgn[b, pl.ds(c * 16, 16)]
            for r in range(rows):
                v = jnp.maximum(buf[r, pl.ds(c * 16, 16)] * sg, 0.0)
                if r % 4 == 0:
                    a0 = a0 + v
                elif r % 4 == 1:
                    a1 = a1 + v
                elif r % 4 == 2:
                    a2 = a2 + v
                else:
                    a3 = a3 + v
            return (a0, a1, a2, a3)
        accs = lax.fori_loop(0, n // 16, cbody, accs)
        if i % halves == halves - 1:
            p_local = i // halves
            outv[pl.ds(p_local * 16, 16)] = (accs[0] + accs[1]) + (accs[2] + accs[3])
            accs = (jnp.zeros((16,), jnp.float32),) * 4

    pltpu.sync_copy(outv, out_hbm.at[pl.ds(wid * ppw * 16, ppw * 16)])


def _tail_body(a_ref, xr_ref, wt_ref, biasr_ref, out_ref, *,
               num_patches, patch):
    P = num_patches
    a_full = a_ref[:]
    T = xr_ref[:] * wt_ref[:]
    N = T.shape[-1]
    n_iota = jax.lax.broadcasted_iota(jnp.int32, (N, P), 0)
    p_iota = jax.lax.broadcasted_iota(jnp.int32, (N, P), 1)
    ind = ((n_iota // patch) == p_iota).astype(jnp.float32)
    pd = jnp.dot(T, ind, preferred_element_type=jnp.float32)

    a2 = a_full
    ap = a2[:, :, None]
    aq = a2[:, None, :]
    qi = jax.lax.broadcasted_iota(jnp.int32, (a2.shape[0], P, P), 2)
    pi = jax.lax.broadcasted_iota(jnp.int32, (a2.shape[0], P, P), 1)
    beats = (aq > ap) | ((aq == ap) & (qi < pi))
    rank = jnp.sum(beats.astype(jnp.float32), axis=-1)
    wgt = jnp.float32(P) - rank

    S = jnp.sum(wgt * pd, axis=-1, keepdims=True)
    biasr = biasr_ref[:]
    inf0 = jnp.sum(pd, axis=-1, keepdims=True) + biasr
    dx = jnp.float32(1.0 / (P + 2))
    out_ref[:] = dx * (0.5 * (1.0 + biasr / inf0)
                       + (jnp.float32(P) * inf0 - S) / inf0)


def kernel(x, attr, mask, W, bias):
    B, M, F, N = attr.shape
    patch = int(F * 0.0625)
    P = F // patch
    BM = B * M

    attr3 = attr.reshape(BM, F, N)

    M_BLK = 2
    HALF = BM // 2
    a0, a1 = pl.pallas_call(
        functools.partial(_tc_reduce_body, num_patches=P, patch=patch,
                          m_blk=M_BLK, mm=M, half=HALF),
        grid=(HALF // M_BLK,),
        in_specs=[
            pl.BlockSpec((B, N), lambda i: (0, 0)),
            pl.BlockSpec((M_BLK, F, N), lambda i: (i, 0, 0)),
            pl.BlockSpec((M_BLK, F, N), lambda i: (i + HALF // M_BLK, 0, 0)),
        ],
        out_specs=[
            pl.BlockSpec((M_BLK, 1, P), lambda i: (i, 0, 0)),
            pl.BlockSpec((M_BLK, 1, P), lambda i: (i, 0, 0)),
        ],
        out_shape=[
            jax.ShapeDtypeStruct((HALF, 1, P), jnp.float32),
            jax.ShapeDtypeStruct((HALF, 1, P), jnp.float32),
        ],
    )(x, attr3, attr3)

    a2 = jnp.concatenate([a0.reshape(HALF, P), a1.reshape(HALF, P)], axis=0)

    xr = jnp.repeat(x, M, axis=0)
    wt = jnp.tile(W.T, (B, 1))
    biasr = jnp.tile(bias, B).reshape(BM, 1)

    out_flat = pl.pallas_call(
        functools.partial(_tail_body, num_patches=P, patch=patch),
        out_shape=jax.ShapeDtypeStruct((BM, 1), jnp.float32),
    )(a2, xr, wt, biasr)
    return out_flat.reshape(B, M)


# speedup vs baseline: 1.1866x; 1.0077x over previous
import functools

import jax
import jax.numpy as jnp
from jax import lax
from jax.experimental import pallas as pl
from jax.experimental.pallas import tpu as pltpu
from jax.experimental.pallas import tpu_sc as plsc


def _tc_reduce_body(x_ref, attr0_ref, attr1_ref, a0_ref, a1_ref, *,
                    num_patches, patch, m_blk, mm, half):
    i = pl.program_id(0)
    for k, (attr_ref, a_ref) in enumerate(((attr0_ref, a0_ref),
                                           (attr1_ref, a1_ref))):
        for j in range(m_blk):
            b = (k * half + i * m_blk + j) // mm
            s = jnp.sign(x_ref[b])
            v = jnp.maximum(attr_ref[j] * s[None, :], 0.0)
            psum = v.reshape(num_patches, patch, v.shape[-1]).sum(axis=(1, 2))
            a_ref[j, 0] = psum


def _sc_reduce_body(x_hbm, attr_hbm, out_hbm, xv, sgn, buf0, buf1, outv,
                    sem0, sem1, *, bm_base, mm, num_patches, patch, ppw,
                    rows, n, nb):
    nc = 2
    wid = lax.axis_index("s") * nc + lax.axis_index("c")
    p0 = wid * ppw

    pltpu.sync_copy(x_hbm, xv)
    for bi in range(nb):
        def sgn_body(c, carry):
            sgn[bi, pl.ds(c * 16, 16)] = jnp.sign(xv[bi, pl.ds(c * 16, 16)])
            return carry
        lax.fori_loop(0, n // 16, sgn_body, jnp.float32(0.0))

    halves = patch // rows
    bufs = (buf0, buf1)
    sems = (sem0, sem1)
    total = ppw * halves

    def chunk_src(i):
        pf = p0 + (i // halves)
        bm = bm_base + pf // num_patches
        f0 = (pf % num_patches) * patch + (i % halves) * rows
        return attr_hbm.at[bm, pl.ds(f0, rows)]

    copies = [None, None]
    copies[0] = pltpu.async_copy(chunk_src(0), bufs[0], sems[0])
    accs = (jnp.zeros((16,), jnp.float32),) * 4
    for i in range(total):
        if i + 1 < total:
            copies[(i + 1) % 2] = pltpu.async_copy(
                chunk_src(i + 1), bufs[(i + 1) % 2], sems[(i + 1) % 2])
        copies[i % 2].wait()
        buf = bufs[i % 2]
        pf = p0 + (i // halves)
        b = (bm_base + pf // num_patches) // mm

        def cbody(c, accs):
            a0, a1, a2, a3 = accs
            sg = sgn[b, pl.ds(c * 16, 16)]
            for r in range(rows):
                v = jnp.maximum(buf[r, pl.ds(c * 16, 16)] * sg, 0.0)
                if r % 4 == 0:
                    a0 = a0 + v
                elif r % 4 == 1:
                    a1 = a1 + v
                elif r % 4 == 2:
                    a2 = a2 + v
                else:
                    a3 = a3 + v
            return (a0, a1, a2, a3)
        accs = lax.fori_loop(0, n // 16, cbody, accs)
        if i % halves == halves - 1:
            p_local = i // halves
            outv[pl.ds(p_local * 16, 16)] = (accs[0] + accs[1]) + (accs[2] + accs[3])
            accs = (jnp.zeros((16,), jnp.float32),) * 4

    pltpu.sync_copy(outv, out_hbm.at[pl.ds(wid * ppw * 16, ppw * 16)])


def _tail_body(a_ref, xr_ref, wt_ref, biasr_ref, out_ref, *,
               num_patches, patch):
    P = num_patches
    a_full = a_ref[:]
    T = xr_ref[:] * wt_ref[:]
    N = T.shape[-1]
    n_iota = jax.lax.broadcasted_iota(jnp.int32, (N, P), 0)
    p_iota = jax.lax.broadcasted_iota(jnp.int32, (N, P), 1)
    ind = ((n_iota // patch) == p_iota).astype(jnp.float32)
    pd = jnp.dot(T, ind, preferred_element_type=jnp.float32)

    a2 = a_full
    ap = a2[:, :, None]
    aq = a2[:, None, :]
    qi = jax.lax.broadcasted_iota(jnp.int32, (a2.shape[0], P, P), 2)
    pi = jax.lax.broadcasted_iota(jnp.int32, (a2.shape[0], P, P), 1)
    beats = (aq > ap) | ((aq == ap) & (qi < pi))
    rank = jnp.sum(beats.astype(jnp.float32), axis=-1)
    wgt = jnp.float32(P) - rank

    S = jnp.sum(wgt * pd, axis=-1, keepdims=True)
    biasr = biasr_ref[:]
    inf0 = jnp.sum(pd, axis=-1, keepdims=True) + biasr
    dx = jnp.float32(1.0 / (P + 2))
    out_ref[:] = dx * (0.5 * (1.0 + biasr / inf0)
                       + (jnp.float32(P) * inf0 - S) / inf0)


def kernel(x, attr, mask, W, bias):
    B, M, F, N = attr.shape
    patch = int(F * 0.0625)
    P = F // patch
    BM = B * M

    attr3 = attr.reshape(BM, F, N)

    M_BLK = 1
    HALF = BM // 2
    a0, a1 = pl.pallas_call(
        functools.partial(_tc_reduce_body, num_patches=P, patch=patch,
                          m_blk=M_BLK, mm=M, half=HALF),
        grid=(HALF // M_BLK,),
        in_specs=[
            pl.BlockSpec((B, N), lambda i: (0, 0)),
            pl.BlockSpec((M_BLK, F, N), lambda i: (i, 0, 0)),
            pl.BlockSpec((M_BLK, F, N), lambda i: (i + HALF // M_BLK, 0, 0)),
        ],
        out_specs=[
            pl.BlockSpec((M_BLK, 1, P), lambda i: (i, 0, 0)),
            pl.BlockSpec((M_BLK, 1, P), lambda i: (i, 0, 0)),
        ],
        out_shape=[
            jax.ShapeDtypeStruct((HALF, 1, P), jnp.float32),
            jax.ShapeDtypeStruct((HALF, 1, P), jnp.float32),
        ],
    )(x, attr3, attr3)

    a2 = jnp.concatenate([a0.reshape(HALF, P), a1.reshape(HALF, P)], axis=0)

    xr = jnp.repeat(x, M, axis=0)
    wt = jnp.tile(W.T, (B, 1))
    biasr = jnp.tile(bias, B).reshape(BM, 1)

    out_flat = pl.pallas_call(
        functools.partial(_tail_body, num_patches=P, patch=patch),
        out_shape=jax.ShapeDtypeStruct((BM, 1), jnp.float32),
    )(a2, xr, wt, biasr)
    return out_flat.reshape(B, M)


# speedup vs baseline: 1.2247x; 1.0321x over previous
import functools

import jax
import jax.numpy as jnp
from jax import lax
from jax.experimental import pallas as pl
from jax.experimental.pallas import tpu as pltpu
from jax.experimental.pallas import tpu_sc as plsc


def _tc_reduce_body(x_ref, attr_ref, a_ref, *, num_patches, patch, m_blk, mm):
    i = pl.program_id(0)
    for j in range(m_blk):
        b = (i * m_blk + j) // mm
        s = jnp.sign(x_ref[b])
        v = jnp.maximum(attr_ref[j] * s[None, :], 0.0)
        psum = v.reshape(num_patches, patch, v.shape[-1]).sum(axis=(1, 2))
        a_ref[j, 0] = psum


def _sc_reduce_body(x_hbm, attr_hbm, out_hbm, xv, sgn, buf0, buf1, outv,
                    sem0, sem1, *, bm_base, mm, num_patches, patch, ppw,
                    rows, n, nb):
    nc = 2
    wid = lax.axis_index("s") * nc + lax.axis_index("c")
    p0 = wid * ppw

    pltpu.sync_copy(x_hbm, xv)
    for bi in range(nb):
        def sgn_body(c, carry):
            sgn[bi, pl.ds(c * 16, 16)] = jnp.sign(xv[bi, pl.ds(c * 16, 16)])
            return carry
        lax.fori_loop(0, n // 16, sgn_body, jnp.float32(0.0))

    halves = patch // rows
    bufs = (buf0, buf1)
    sems = (sem0, sem1)
    total = ppw * halves

    def chunk_src(i):
        pf = p0 + (i // halves)
        bm = bm_base + pf // num_patches
        f0 = (pf % num_patches) * patch + (i % halves) * rows
        return attr_hbm.at[bm, pl.ds(f0, rows)]

    copies = [None, None]
    copies[0] = pltpu.async_copy(chunk_src(0), bufs[0], sems[0])
    accs = (jnp.zeros((16,), jnp.float32),) * 4
    for i in range(total):
        if i + 1 < total:
            copies[(i + 1) % 2] = pltpu.async_copy(
                chunk_src(i + 1), bufs[(i + 1) % 2], sems[(i + 1) % 2])
        copies[i % 2].wait()
        buf = bufs[i % 2]
        pf = p0 + (i // halves)
        b = (bm_base + pf // num_patches) // mm

        def cbody(c, accs):
            a0, a1, a2, a3 = accs
            sg = sgn[b, pl.ds(c * 16, 16)]
            for r in range(rows):
                v = jnp.maximum(buf[r, pl.ds(c * 16, 16)] * sg, 0.0)
                if r % 4 == 0:
                    a0 = a0 + v
                elif r % 4 == 1:
                    a1 = a1 + v
                elif r % 4 == 2:
                    a2 = a2 + v
                else:
                    a3 = a3 + v
            return (a0, a1, a2, a3)
        accs = lax.fori_loop(0, n // 16, cbody, accs)
        if i % halves == halves - 1:
            p_local = i // halves
            outv[pl.ds(p_local * 16, 16)] = (accs[0] + accs[1]) + (accs[2] + accs[3])
            accs = (jnp.zeros((16,), jnp.float32),) * 4

    pltpu.sync_copy(outv, out_hbm.at[pl.ds(wid * ppw * 16, ppw * 16)])


def _tail_body(a_ref, xr_ref, wt_ref, biasr_ref, out_ref, *,
               num_patches, patch):
    P = num_patches
    a_full = a_ref[:]
    T = xr_ref[:] * wt_ref[:]
    N = T.shape[-1]
    n_iota = jax.lax.broadcasted_iota(jnp.int32, (N, P), 0)
    p_iota = jax.lax.broadcasted_iota(jnp.int32, (N, P), 1)
    ind = ((n_iota // patch) == p_iota).astype(jnp.float32)
    pd = jnp.dot(T, ind, preferred_element_type=jnp.float32)

    a2 = a_full
    ap = a2[:, :, None]
    aq = a2[:, None, :]
    qi = jax.lax.broadcasted_iota(jnp.int32, (a2.shape[0], P, P), 2)
    pi = jax.lax.broadcasted_iota(jnp.int32, (a2.shape[0], P, P), 1)
    beats = (aq > ap) | ((aq == ap) & (qi < pi))
    rank = jnp.sum(beats.astype(jnp.float32), axis=-1)
    wgt = jnp.float32(P) - rank

    S = jnp.sum(wgt * pd, axis=-1, keepdims=True)
    biasr = biasr_ref[:]
    inf0 = jnp.sum(pd, axis=-1, keepdims=True) + biasr
    dx = jnp.float32(1.0 / (P + 2))
    out_ref[:] = dx * (0.5 * (1.0 + biasr / inf0)
                       + (jnp.float32(P) * inf0 - S) / inf0)


def kernel(x, attr, mask, W, bias):
    B, M, F, N = attr.shape
    patch = int(F * 0.0625)
    P = F // patch
    BM = B * M

    attr3 = attr.reshape(BM, F, N)

    M_BLK = 2
    a = pl.pallas_call(
        functools.partial(_tc_reduce_body, num_patches=P, patch=patch,
                          m_blk=M_BLK, mm=M),
        grid=(BM // M_BLK,),
        in_specs=[
            pl.BlockSpec((B, N), lambda i: (0, 0)),
            pl.BlockSpec((M_BLK, F, N), lambda i: (i, 0, 0)),
        ],
        out_specs=pl.BlockSpec((M_BLK, 1, P), lambda i: (i, 0, 0)),
        out_shape=jax.ShapeDtypeStruct((BM, 1, P), jnp.float32),
    )(x, attr3)

    a2 = a.reshape(BM, P)

    xr = jnp.repeat(x, M, axis=0)
    wt = jnp.tile(W.T, (B, 1))
    biasr = jnp.tile(bias, B).reshape(BM, 1)

    out_flat = pl.pallas_call(
        functools.partial(_tail_body, num_patches=P, patch=patch),
        out_shape=jax.ShapeDtypeStruct((BM, 1), jnp.float32),
    )(a2, xr, wt, biasr)
    return out_flat.reshape(B, M)
